# Initial kernel scaffold; baseline (speedup 1.0000x reference)
import jax
import jax.numpy as jnp
from jax.experimental import pallas as pl


def kernel(x, Win, bin_, Wout, bout):
    raise NotImplementedError("write your pallas kernel here")


# speedup vs baseline: 2.2611x; 2.2611x over previous
import jax
import jax.numpy as jnp
import numpy as np
from jax.experimental import pallas as pl
from jax.experimental.pallas import tpu as pltpu

G = 2
R = 2
DIM = 1024
DPG = DIM // G
CD = 4
NLEV = 5.0
N_IND = 625
EPS = 1e-5
HALF_L = (NLEV - 1.0) * (1.0 + 1e-3) / 2.0
HW = 2.0

TT = 512


def _gfsq_kernel(x_ref, win_ref, bin_ref, wout_ref, bout_ref,
                 feat_ref, ind_ref, perp_ref, hist_ref):
    b = pl.program_id(1)
    t = pl.program_id(2)
    nb = pl.num_programs(1)
    nt = pl.num_programs(2)

    @pl.when(jnp.logical_and(b == 0, t == 0))
    def _init():
        hist_ref[...] = jnp.zeros_like(hist_ref)

    xb = x_ref[0]
    win = win_ref[0]
    z = jnp.dot(win, xb, preferred_element_type=jnp.float32)
    z = z + bin_ref[0].reshape(CD, 1)

    r0 = jnp.round(jnp.tanh(z) * HALF_L)
    c0 = r0 * (1.0 / HW)
    res = z - c0
    r1 = jnp.round(jnp.tanh(res * 4.0) * HALF_L)
    c1 = r1 * (1.0 / HW)
    q = c0 + c1 * 0.25

    out = jnp.dot(wout_ref[0], q, preferred_element_type=jnp.float32)
    feat_ref[0] = out + bout_ref[0].reshape(DPG, 1)

    iota25 = jax.lax.broadcasted_iota(jnp.int32, (25, TT), 0)
    idx_rows = []
    for s, rs in enumerate((r0, r1)):
        d = rs + HW
        p01 = (d[0:1] + 5.0 * d[1:2]).astype(jnp.int32)
        p23 = (d[2:3] + 5.0 * d[3:4]).astype(jnp.int32)
        idx_rows.append(p01 + 25 * p23)
        oh01 = (iota25 == p01).astype(jnp.float32)
        oh23 = (iota25 == p23).astype(jnp.float32)
        joint = jax.lax.dot_general(
            oh23, oh01, (((1,), (1,)), ((), ())),
            preferred_element_type=jnp.float32)
        hist_ref[s] = hist_ref[s] + joint

    ind_ref[0, 0] = jnp.concatenate(idx_rows, axis=0)

    @pl.when(jnp.logical_and(b == nb - 1, t == nt - 1))
    def _finish():
        cnt = hist_ref[...]
        em = cnt / jnp.float32(nb * nt * TT)
        em = em / (jnp.sum(em, axis=(1, 2), keepdims=True) + EPS)
        ent = -jnp.sum(em * jnp.log(em + EPS), axis=(1, 2))
        perp_ref[0, 0] = jnp.exp(ent)


def kernel(x, Win, bin_, Wout, bout):
    B, _, T = x.shape
    nt = T // TT
    grid = (G, B, nt)

    feat, ind, perp = pl.pallas_call(
        _gfsq_kernel,
        grid=grid,
        in_specs=[
            pl.BlockSpec((1, DPG, TT), lambda g, b, t: (b, g, t)),
            pl.BlockSpec((1, CD, DPG), lambda g, b, t: (g, 0, 0)),
            pl.BlockSpec((1, 1, CD), lambda g, b, t: (g, 0, 0)),
            pl.BlockSpec((1, DPG, CD), lambda g, b, t: (g, 0, 0)),
            pl.BlockSpec((1, 1, DPG), lambda g, b, t: (g, 0, 0)),
        ],
        out_specs=[
            pl.BlockSpec((1, DPG, TT), lambda g, b, t: (b, g, t)),
            pl.BlockSpec((1, 1, R, TT), lambda g, b, t: (b, g, 0, t)),
            pl.BlockSpec((1, 1, R), lambda g, b, t: (g, 0, 0)),
        ],
        out_shape=[
            jax.ShapeDtypeStruct((B, DIM, T), jnp.float32),
            jax.ShapeDtypeStruct((B, G, R, T), jnp.int32),
            jax.ShapeDtypeStruct((G, 1, R), jnp.float32),
        ],
        scratch_shapes=[pltpu.VMEM((R, 25, 25), jnp.float32)],
    )(x, Win, bin_.reshape(G, 1, CD), Wout, bout.reshape(G, 1, DPG))

    ind_out = ind.reshape(B, G * R, T)
    perplexity = perp.reshape(G * R)
    zeros = jnp.zeros_like(perplexity)
    return (zeros, feat, perplexity, ind_out)


# speedup vs baseline: 3.1074x; 1.3743x over previous
import jax
import jax.numpy as jnp
import numpy as np
from jax.experimental import pallas as pl
from jax.experimental.pallas import tpu as pltpu

G = 2
R = 2
DIM = 1024
DPG = DIM // G
CD = 4
NLEV = 5.0
N_IND = 625
EPS = 1e-5
HALF_L = (NLEV - 1.0) * (1.0 + 1e-3) / 2.0
HW = 2.0

TT = 1024


def _gfsq_kernel(x_ref, win_ref, bin_ref, wout_ref, bout_ref,
                 feat_ref, ind_ref, perp_ref, hist_ref):
    b = pl.program_id(1)
    t = pl.program_id(2)
    nb = pl.num_programs(1)
    nt = pl.num_programs(2)

    @pl.when(jnp.logical_and(b == 0, t == 0))
    def _init():
        hist_ref[...] = jnp.zeros_like(hist_ref)

    xb = x_ref[0]
    win = win_ref[0]
    z = jnp.dot(win, xb, preferred_element_type=jnp.float32)
    z = z + bin_ref[0].reshape(CD, 1)

    r0 = jnp.round(jnp.tanh(z) * HALF_L)
    c0 = r0 * (1.0 / HW)
    res = z - c0
    r1 = jnp.round(jnp.tanh(res * 4.0) * HALF_L)
    c1 = r1 * (1.0 / HW)
    q = c0 + c1 * 0.25

    out = jnp.dot(wout_ref[0], q, preferred_element_type=jnp.float32)
    feat_ref[0] = out + bout_ref[0].reshape(DPG, 1)

    iota25 = jax.lax.broadcasted_iota(jnp.int32, (25, TT), 0)
    idx_rows = []
    for s, rs in enumerate((r0, r1)):
        d = rs + HW
        p01 = (d[0:1] + 5.0 * d[1:2]).astype(jnp.int32)
        p23 = (d[2:3] + 5.0 * d[3:4]).astype(jnp.int32)
        idx_rows.append(p01 + 25 * p23)
        oh01 = (iota25 == p01).astype(jnp.float32)
        oh23 = (iota25 == p23).astype(jnp.float32)
        joint = jax.lax.dot_general(
            oh23, oh01, (((1,), (1,)), ((), ())),
            preferred_element_type=jnp.float32)
        hist_ref[s] = hist_ref[s] + joint

    ind_ref[0, 0] = jnp.concatenate(idx_rows, axis=0)

    @pl.when(jnp.logical_and(b == nb - 1, t == nt - 1))
    def _finish():
        cnt = hist_ref[...]
        em = cnt / jnp.float32(nb * nt * TT)
        em = em / (jnp.sum(em, axis=(1, 2), keepdims=True) + EPS)
        ent = -jnp.sum(em * jnp.log(em + EPS), axis=(1, 2))
        perp_ref[0, 0] = jnp.exp(ent)


def kernel(x, Win, bin_, Wout, bout):
    B, _, T = x.shape
    nt = T // TT
    grid = (G, B, nt)

    feat, ind, perp = pl.pallas_call(
        _gfsq_kernel,
        grid=grid,
        in_specs=[
            pl.BlockSpec((1, DPG, TT), lambda g, b, t: (b, g, t)),
            pl.BlockSpec((1, CD, DPG), lambda g, b, t: (g, 0, 0)),
            pl.BlockSpec((1, 1, CD), lambda g, b, t: (g, 0, 0)),
            pl.BlockSpec((1, DPG, CD), lambda g, b, t: (g, 0, 0)),
            pl.BlockSpec((1, 1, DPG), lambda g, b, t: (g, 0, 0)),
        ],
        out_specs=[
            pl.BlockSpec((1, DPG, TT), lambda g, b, t: (b, g, t)),
            pl.BlockSpec((1, 1, R, TT), lambda g, b, t: (b, g, 0, t)),
            pl.BlockSpec((1, 1, R), lambda g, b, t: (g, 0, 0)),
        ],
        out_shape=[
            jax.ShapeDtypeStruct((B, DIM, T), jnp.float32),
            jax.ShapeDtypeStruct((B, G, R, T), jnp.int32),
            jax.ShapeDtypeStruct((G, 1, R), jnp.float32),
        ],
        scratch_shapes=[pltpu.VMEM((R, 25, 25), jnp.float32)],
    )(x, Win, bin_.reshape(G, 1, CD), Wout, bout.reshape(G, 1, DPG))

    ind_out = ind.reshape(B, G * R, T)
    perplexity = perp.reshape(G * R)
    zeros = jnp.zeros_like(perplexity)
    return (zeros, feat, perplexity, ind_out)


# speedup vs baseline: 3.5989x; 1.1582x over previous
import jax
import jax.numpy as jnp
import numpy as np
from jax.experimental import pallas as pl
from jax.experimental.pallas import tpu as pltpu

G = 2
R = 2
DIM = 1024
DPG = DIM // G
CD = 4
NLEV = 5.0
N_IND = 625
EPS = 1e-5
HALF_L = (NLEV - 1.0) * (1.0 + 1e-3) / 2.0
HW = 2.0

TT = 2048


def _gfsq_kernel(x_ref, win_ref, bin_ref, wout_ref, bout_ref,
                 feat_ref, ind_ref, perp_ref, hist_ref):
    b = pl.program_id(1)
    t = pl.program_id(2)
    nb = pl.num_programs(1)
    nt = pl.num_programs(2)

    @pl.when(jnp.logical_and(b == 0, t == 0))
    def _init():
        hist_ref[...] = jnp.zeros_like(hist_ref)

    xb = x_ref[0]
    win = win_ref[0]
    z = jnp.dot(win, xb, preferred_element_type=jnp.float32)
    z = z + bin_ref[0].reshape(CD, 1)

    r0 = jnp.round(jnp.tanh(z) * HALF_L)
    c0 = r0 * (1.0 / HW)
    res = z - c0
    r1 = jnp.round(jnp.tanh(res * 4.0) * HALF_L)
    c1 = r1 * (1.0 / HW)
    q = c0 + c1 * 0.25

    out = jnp.dot(wout_ref[0], q, preferred_element_type=jnp.float32)
    feat_ref[0] = out + bout_ref[0].reshape(DPG, 1)

    iota25 = jax.lax.broadcasted_iota(jnp.int32, (25, TT), 0)
    idx_rows = []
    for s, rs in enumerate((r0, r1)):
        d = rs + HW
        p01 = (d[0:1] + 5.0 * d[1:2]).astype(jnp.int32)
        p23 = (d[2:3] + 5.0 * d[3:4]).astype(jnp.int32)
        idx_rows.append(p01 + 25 * p23)
        oh01 = (iota25 == p01).astype(jnp.float32)
        oh23 = (iota25 == p23).astype(jnp.float32)
        joint = jax.lax.dot_general(
            oh23, oh01, (((1,), (1,)), ((), ())),
            preferred_element_type=jnp.float32)
        hist_ref[s] = hist_ref[s] + joint

    ind_ref[0, 0] = jnp.concatenate(idx_rows, axis=0)

    @pl.when(jnp.logical_and(b == nb - 1, t == nt - 1))
    def _finish():
        cnt = hist_ref[...]
        em = cnt / jnp.float32(nb * nt * TT)
        em = em / (jnp.sum(em, axis=(1, 2), keepdims=True) + EPS)
        ent = -jnp.sum(em * jnp.log(em + EPS), axis=(1, 2))
        perp_ref[0, 0] = jnp.exp(ent)


def kernel(x, Win, bin_, Wout, bout):
    B, _, T = x.shape
    nt = T // TT
    grid = (G, B, nt)

    feat, ind, perp = pl.pallas_call(
        _gfsq_kernel,
        grid=grid,
        in_specs=[
            pl.BlockSpec((1, DPG, TT), lambda g, b, t: (b, g, t)),
            pl.BlockSpec((1, CD, DPG), lambda g, b, t: (g, 0, 0)),
            pl.BlockSpec((1, 1, CD), lambda g, b, t: (g, 0, 0)),
            pl.BlockSpec((1, DPG, CD), lambda g, b, t: (g, 0, 0)),
            pl.BlockSpec((1, 1, DPG), lambda g, b, t: (g, 0, 0)),
        ],
        out_specs=[
            pl.BlockSpec((1, DPG, TT), lambda g, b, t: (b, g, t)),
            pl.BlockSpec((1, 1, R, TT), lambda g, b, t: (b, g, 0, t)),
            pl.BlockSpec((1, 1, R), lambda g, b, t: (g, 0, 0)),
        ],
        out_shape=[
            jax.ShapeDtypeStruct((B, DIM, T), jnp.float32),
            jax.ShapeDtypeStruct((B, G, R, T), jnp.int32),
            jax.ShapeDtypeStruct((G, 1, R), jnp.float32),
        ],
        scratch_shapes=[pltpu.VMEM((R, 25, 25), jnp.float32)],
    )(x, Win, bin_.reshape(G, 1, CD), Wout, bout.reshape(G, 1, DPG))

    ind_out = ind.reshape(B, G * R, T)
    perplexity = perp.reshape(G * R)
    zeros = jnp.zeros_like(perplexity)
    return (zeros, feat, perplexity, ind_out)


# speedup vs baseline: 3.7096x; 1.0308x over previous
import jax
import jax.numpy as jnp
import numpy as np
from jax.experimental import pallas as pl
from jax.experimental.pallas import tpu as pltpu

G = 2
R = 2
DIM = 1024
DPG = DIM // G
CD = 4
NLEV = 5.0
N_IND = 625
EPS = 1e-5
HALF_L = (NLEV - 1.0) * (1.0 + 1e-3) / 2.0
HW = 2.0

TT = 4096


def _gfsq_kernel(x_ref, win_ref, bin_ref, wout_ref, bout_ref,
                 feat_ref, ind_ref, perp_ref, hist_ref):
    b = pl.program_id(1)
    t = pl.program_id(2)
    nb = pl.num_programs(1)
    nt = pl.num_programs(2)

    @pl.when(jnp.logical_and(b == 0, t == 0))
    def _init():
        hist_ref[...] = jnp.zeros_like(hist_ref)

    xb = x_ref[0]
    win = win_ref[0]
    z = jnp.dot(win, xb, preferred_element_type=jnp.float32)
    z = z + bin_ref[0].reshape(CD, 1)

    r0 = jnp.round(jnp.tanh(z) * HALF_L)
    c0 = r0 * (1.0 / HW)
    res = z - c0
    r1 = jnp.round(jnp.tanh(res * 4.0) * HALF_L)
    c1 = r1 * (1.0 / HW)
    q = c0 + c1 * 0.25

    out = jnp.dot(wout_ref[0], q, preferred_element_type=jnp.float32)
    feat_ref[0] = out + bout_ref[0].reshape(DPG, 1)

    iota25 = jax.lax.broadcasted_iota(jnp.int32, (25, TT), 0)
    idx_rows = []
    for s, rs in enumerate((r0, r1)):
        d = rs + HW
        p01 = (d[0:1] + 5.0 * d[1:2]).astype(jnp.int32)
        p23 = (d[2:3] + 5.0 * d[3:4]).astype(jnp.int32)
        idx_rows.append(p01 + 25 * p23)
        oh01 = (iota25 == p01).astype(jnp.float32)
        oh23 = (iota25 == p23).astype(jnp.float32)
        joint = jax.lax.dot_general(
            oh23, oh01, (((1,), (1,)), ((), ())),
            preferred_element_type=jnp.float32)
        hist_ref[s] = hist_ref[s] + joint

    ind_ref[0, 0] = jnp.concatenate(idx_rows, axis=0)

    @pl.when(jnp.logical_and(b == nb - 1, t == nt - 1))
    def _finish():
        cnt = hist_ref[...]
        em = cnt / jnp.float32(nb * nt * TT)
        em = em / (jnp.sum(em, axis=(1, 2), keepdims=True) + EPS)
        ent = -jnp.sum(em * jnp.log(em + EPS), axis=(1, 2))
        perp_ref[0, 0] = jnp.exp(ent)


def kernel(x, Win, bin_, Wout, bout):
    B, _, T = x.shape
    nt = T // TT
    grid = (G, B, nt)

    feat, ind, perp = pl.pallas_call(
        _gfsq_kernel,
        grid=grid,
        in_specs=[
            pl.BlockSpec((1, DPG, TT), lambda g, b, t: (b, g, t)),
            pl.BlockSpec((1, CD, DPG), lambda g, b, t: (g, 0, 0)),
            pl.BlockSpec((1, 1, CD), lambda g, b, t: (g, 0, 0)),
            pl.BlockSpec((1, DPG, CD), lambda g, b, t: (g, 0, 0)),
            pl.BlockSpec((1, 1, DPG), lambda g, b, t: (g, 0, 0)),
        ],
        out_specs=[
            pl.BlockSpec((1, DPG, TT), lambda g, b, t: (b, g, t)),
            pl.BlockSpec((1, 1, R, TT), lambda g, b, t: (b, g, 0, t)),
            pl.BlockSpec((1, 1, R), lambda g, b, t: (g, 0, 0)),
        ],
        out_shape=[
            jax.ShapeDtypeStruct((B, DIM, T), jnp.float32),
            jax.ShapeDtypeStruct((B, G, R, T), jnp.int32),
            jax.ShapeDtypeStruct((G, 1, R), jnp.float32),
        ],
        scratch_shapes=[pltpu.VMEM((R, 25, 25), jnp.float32)],
    )(x, Win, bin_.reshape(G, 1, CD), Wout, bout.reshape(G, 1, DPG))

    ind_out = ind.reshape(B, G * R, T)
    perplexity = perp.reshape(G * R)
    zeros = jnp.zeros_like(perplexity)
    return (zeros, feat, perplexity, ind_out)
